# baseline (speedup 1.0000x reference)
import jax
import jax.numpy as jnp
import numpy as np
from jax.experimental import pallas as pl


def _prelu(x, a):
    return jnp.where(x >= 0, x, a * x)


def _mpnn(x, src, dst, gid, p):
    N = x.shape[0]
    H = 128
    B = 64
    h0 = jax.nn.relu(x @ p['Wp'] + p['bp'])
    hidden = h0
    w = jax.nn.sigmoid(jnp.ones((src.shape[0],), jnp.float32))
    deg_out = jax.ops.segment_sum(w, src, num_segments=N)
    deg_in = jax.ops.segment_sum(w, dst, num_segments=N)
    wn = w * (deg_out[src] ** -0.5) * (deg_in[dst] ** -0.5)
    nf = h0
    for _ in range(3):
        fstack = [nf]
        cur = nf
        for _k in range(2):
            cur = jax.ops.segment_sum(cur[src] * wn[:, None], dst, num_segments=N)
            fstack.append(cur)
        t = jax.nn.relu(jnp.concatenate(fstack, axis=-1) @ p['Wtag'] + p['btag'])
        cat = jnp.concatenate([t, hidden], axis=-1)
        nf = cat @ p['Wg1'] + p['bg1']
        hidden = cat @ p['Wg2'] + p['bg2']
    feat = jnp.concatenate([h0, nf], axis=1)
    d = feat.shape[1]
    hC = jnp.zeros((B, d), jnp.float32)
    cC = jnp.zeros((B, d), jnp.float32)
    q_star = jnp.zeros((B, 2 * d), jnp.float32)
    for _ in range(3):
        gates = q_star @ p['W_ih'].T + p['b_ih'] + hC @ p['W_hh'].T + p['b_hh']
        i, f, g, o = jnp.split(gates, 4, axis=-1)
        cC = jax.nn.sigmoid(f) * cC + jax.nn.sigmoid(i) * jnp.tanh(g)
        hC = jax.nn.sigmoid(o) * jnp.tanh(cC)
        q = hC
        e = jnp.sum(feat * q[gid], axis=-1)
        emax = jax.ops.segment_max(e, gid, num_segments=B)
        ee = jnp.exp(e - emax[gid])
        den = jax.ops.segment_sum(ee, gid, num_segments=B)
        alpha = ee / den[gid]
        r = jax.ops.segment_sum(feat * alpha[:, None], gid, num_segments=B)
        q_star = jnp.concatenate([q, r], axis=-1)
    return _prelu(q_star @ p['Wsp'] + p['bsp'], p['asp'])


def kernel(x_r, edge_index_r, edge_attr_r, graph_ids_r,
           x_p, edge_index_p, edge_attr_p, graph_ids_p,
           Wp, bp, Wtag, btag, Wg1, bg1, Wg2, bg2,
           W_ih, W_hh, b_ih, b_hh, Wsp, bsp, asp,
           Wp1, bp1, ap1, Wp2, bp2, ap2, Wp3, bp3):
    p = dict(Wp=Wp, bp=bp, Wtag=Wtag, btag=btag, Wg1=Wg1, bg1=bg1, Wg2=Wg2, bg2=bg2,
             W_ih=W_ih, W_hh=W_hh, b_ih=b_ih, b_hh=b_hh, Wsp=Wsp, bsp=bsp, asp=asp)
    gr = _mpnn(x_r, edge_index_r[0], edge_index_r[1], graph_ids_r, p)
    gp = _mpnn(x_p, edge_index_p[0], edge_index_p[1], graph_ids_p, p)
    cat = jnp.concatenate([gr, gp], axis=1)
    h1 = _prelu(cat @ Wp1 + bp1, ap1)
    h2 = _prelu(h1 @ Wp2 + bp2, ap2)
    out = h2 @ Wp3 + bp3
    return (out[:, 0], out[:, 1])


# speedup vs baseline: 3.1042x; 3.1042x over previous
import functools
import math

import jax
import jax.numpy as jnp
from jax import lax
from jax.experimental import pallas as pl
from jax.experimental.pallas import tpu as pltpu
from jax.experimental.pallas import tpu_sc as plsc

N = 10000
NP = 10240
E = 320000
EP = 327680
H = 128
B = 64
NT = 16
NC = 2
CHUNK = 128
EPT = EP // NT
NCHUNK = EPT // CHUNK
RPT = NP // NT
WSIG = 1.0 / (1.0 + math.exp(-1.0))

def _mesh():
    return plsc.VectorSubcoreMesh(
        core_axis_name="c", subcore_axis_name="s", num_cores=NC, num_subcores=NT)


def _sc_prop(u, esrc_g, edst, zeros128):
    @functools.partial(
        pl.kernel,
        out_type=jax.ShapeDtypeStruct((2 * NP, H), jnp.float32),
        mesh=_mesh(),
        scratch_types=[
            pltpu.MemorySpace.VMEM_SHARED((NP, H), jnp.float32),
            pltpu.MemorySpace.VMEM((CHUNK, H), jnp.float32),
            pltpu.MemorySpace.VMEM((CHUNK,), jnp.int32),
            pltpu.MemorySpace.VMEM((CHUNK,), jnp.int32),
            pltpu.SemaphoreType.DMA,
        ],
    )
    def prop_kernel(u_hbm, esrc_hbm, edst_hbm, zeros_hbm, y_hbm,
                    acc, rows, sidx, didx, sem):
        c = lax.axis_index("c")
        s = lax.axis_index("s")
        for b in range(RPT // CHUNK):
            r0 = s * RPT + b * CHUNK
            pltpu.sync_copy(zeros_hbm, acc.at[pl.ds(r0, CHUNK)])
        plsc.subcore_barrier()

        def chunk(j, carry):
            ebase = s * EPT + j * CHUNK
            pltpu.sync_copy(esrc_hbm.at[c, pl.ds(ebase, CHUNK)], sidx)
            pltpu.sync_copy(edst_hbm.at[c, pl.ds(ebase, CHUNK)], didx)
            pltpu.async_copy(u_hbm.at[sidx], rows, sem).wait()
            pltpu.sync_copy(rows, acc.at[didx], add=True)
            return carry

        lax.fori_loop(0, NCHUNK, chunk, 0)
        plsc.subcore_barrier()
        for b in range(RPT // CHUNK):
            r0 = s * RPT + b * CHUNK
            pltpu.sync_copy(acc.at[pl.ds(r0, CHUNK)],
                            y_hbm.at[pl.ds(c * NP + r0, CHUNK)])

    return prop_kernel(u, esrc_g, edst, zeros128)


_RB = 1280
_NB = NP // _RB


def _scales(cnt):
    return jnp.where(cnt > 0, lax.rsqrt(jnp.maximum(cnt, 1e-30) * WSIG), 0.0)


def _tc_prologue(x2, wp, bp, douts):
    def body(x_ref, wp_ref, bp_ref, do_ref, h0_ref, u_ref):
        h0 = jax.nn.relu(
            jnp.dot(x_ref[0], wp_ref[...], preferred_element_type=jnp.float32, precision=lax.Precision.HIGHEST)
            + bp_ref[...])
        t = WSIG * _scales(do_ref[0][:, 0:1])
        h0_ref[0] = h0
        u_ref[0] = h0 * t

    return pl.pallas_call(
        body,
        grid=(2, _NB),
        in_specs=[
            pl.BlockSpec((1, _RB, H), lambda g, i: (g, i, 0)),
            pl.BlockSpec((H, H), lambda g, i: (0, 0)),
            pl.BlockSpec((1, H), lambda g, i: (0, 0)),
            pl.BlockSpec((1, _RB, 16), lambda g, i: (g, i, 0)),
        ],
        out_specs=[
            pl.BlockSpec((1, _RB, H), lambda g, i: (g, i, 0)),
            pl.BlockSpec((1, _RB, H), lambda g, i: (g, i, 0)),
        ],
        out_shape=[
            jax.ShapeDtypeStruct((2, NP, H), jnp.float32),
            jax.ShapeDtypeStruct((2, NP, H), jnp.float32),
        ],
    )(x2, wp, bp, douts)


def _tc_midhop(y1, dins, douts):
    def body(y_ref, di_ref, do_ref, f1_ref, u2_ref):
        s = _scales(di_ref[0][:, 0:1])
        t = WSIG * _scales(do_ref[0][:, 0:1])
        f1 = y_ref[0] * s
        f1_ref[0] = f1
        u2_ref[0] = f1 * t

    return pl.pallas_call(
        body,
        grid=(2, _NB),
        in_specs=[
            pl.BlockSpec((1, _RB, H), lambda g, i: (g, i, 0)),
            pl.BlockSpec((1, _RB, 16), lambda g, i: (g, i, 0)),
            pl.BlockSpec((1, _RB, 16), lambda g, i: (g, i, 0)),
        ],
        out_specs=[
            pl.BlockSpec((1, _RB, H), lambda g, i: (g, i, 0)),
            pl.BlockSpec((1, _RB, H), lambda g, i: (g, i, 0)),
        ],
        out_shape=[
            jax.ShapeDtypeStruct((2, NP, H), jnp.float32),
            jax.ShapeDtypeStruct((2, NP, H), jnp.float32),
        ],
    )(y1, dins, douts)


def _tc_iter_end(y2, nf, f1, hidden, dins, douts, wtag, btag, wg1, bg1, wg2, bg2):
    def body(y_ref, nf_ref, f1_ref, hid_ref, di_ref, do_ref,
             wtag_ref, btag_ref, wg1_ref, bg1_ref, wg2_ref, bg2_ref,
             nf2_ref, hid2_ref, u_ref):
        s = _scales(di_ref[0][:, 0:1])
        t = WSIG * _scales(do_ref[0][:, 0:1])
        f2 = y_ref[0] * s
        fst = jnp.concatenate([nf_ref[0], f1_ref[0], f2], axis=1)
        tt = jax.nn.relu(
            jnp.dot(fst, wtag_ref[...], preferred_element_type=jnp.float32, precision=lax.Precision.HIGHEST)
            + btag_ref[...])
        cat = jnp.concatenate([tt, hid_ref[0]], axis=1)
        nf2 = jnp.dot(cat, wg1_ref[...], preferred_element_type=jnp.float32, precision=lax.Precision.HIGHEST) + bg1_ref[...]
        hid2 = jnp.dot(cat, wg2_ref[...], preferred_element_type=jnp.float32, precision=lax.Precision.HIGHEST) + bg2_ref[...]
        nf2_ref[0] = nf2
        hid2_ref[0] = hid2
        u_ref[0] = nf2 * t

    return pl.pallas_call(
        body,
        grid=(2, _NB),
        in_specs=[
            pl.BlockSpec((1, _RB, H), lambda g, i: (g, i, 0)),
            pl.BlockSpec((1, _RB, H), lambda g, i: (g, i, 0)),
            pl.BlockSpec((1, _RB, H), lambda g, i: (g, i, 0)),
            pl.BlockSpec((1, _RB, H), lambda g, i: (g, i, 0)),
            pl.BlockSpec((1, _RB, 16), lambda g, i: (g, i, 0)),
            pl.BlockSpec((1, _RB, 16), lambda g, i: (g, i, 0)),
            pl.BlockSpec((3 * H, H), lambda g, i: (0, 0)),
            pl.BlockSpec((1, H), lambda g, i: (0, 0)),
            pl.BlockSpec((2 * H, H), lambda g, i: (0, 0)),
            pl.BlockSpec((1, H), lambda g, i: (0, 0)),
            pl.BlockSpec((2 * H, H), lambda g, i: (0, 0)),
            pl.BlockSpec((1, H), lambda g, i: (0, 0)),
        ],
        out_specs=[
            pl.BlockSpec((1, _RB, H), lambda g, i: (g, i, 0)),
            pl.BlockSpec((1, _RB, H), lambda g, i: (g, i, 0)),
            pl.BlockSpec((1, _RB, H), lambda g, i: (g, i, 0)),
        ],
        out_shape=[
            jax.ShapeDtypeStruct((2, NP, H), jnp.float32),
            jax.ShapeDtypeStruct((2, NP, H), jnp.float32),
            jax.ShapeDtypeStruct((2, NP, H), jnp.float32),
        ],
    )(y2, nf, f1, hidden, dins, douts, wtag, btag, wg1, bg1, wg2, bg2)


def _dot3(a, b, dims):
    f32 = jnp.float32
    a_hi = a.astype(jnp.bfloat16).astype(f32)
    a_lo = a - a_hi
    b_hi = b.astype(jnp.bfloat16).astype(f32)
    b_lo = b - b_hi
    d = lambda x, y: lax.dot_general(x, y, dims, preferred_element_type=f32,
                                     precision=lax.Precision.DEFAULT)
    return d(a_hi, b_hi) + d(a_lo, b_hi) + d(a_hi, b_lo)


def _sigm(x):
    return 1.0 / (1.0 + jnp.exp(-x))


def _tc_set2set(feat2, gidb, w_ihT, b_ih, w_hhT, b_hh, wsp, bsp, asp):
    D = 2 * H
    NCK = 8
    CK = NP // NCK

    def body(feat_ref, gidb_ref, wih_ref, bih_ref, whh_ref, bhh_ref,
             wsp_ref, bsp_ref, asp_ref, out_ref, e_ref):
        hC = jnp.zeros((B, D), jnp.float32)
        cC = jnp.zeros((B, D), jnp.float32)
        q_star = jnp.zeros((B, 2 * D), jnp.float32)
        for _ in range(3):
            gates = (_dot3(q_star, wih_ref[...], (((1,), (0,)), ((), ())))
                     + bih_ref[...]
                     + _dot3(hC, whh_ref[...], (((1,), (0,)), ((), ())))
                     + bhh_ref[...])
            ig = gates[:, 0:D]
            fg = gates[:, D:2 * D]
            gg = gates[:, 2 * D:3 * D]
            og = gates[:, 3 * D:4 * D]
            cC = _sigm(fg) * cC + _sigm(ig) * jnp.tanh(gg)
            hC = _sigm(og) * jnp.tanh(cC)
            q = hC
            cols = lax.broadcasted_iota(jnp.int32, (CK, B), 1)

            def p1(k, emax):
                sl = pl.ds(k * CK, CK)
                oh = gidb_ref[0, sl] == cols
                e_k = _dot3(feat_ref[0, sl], q, (((1,), (1,)), ((), ())))
                e_ref[sl] = e_k
                return jnp.maximum(
                    emax, jnp.max(jnp.where(oh, e_k, -3e38), axis=0,
                                  keepdims=True))

            emax = lax.fori_loop(0, NCK, p1, jnp.full((1, B), -3e38, jnp.float32))

            def p2(k, den):
                sl = pl.ds(k * CK, CK)
                oh = gidb_ref[0, sl] == cols
                ee_k = jnp.where(oh, jnp.exp(e_ref[sl] - emax), 0.0)
                e_ref[sl] = ee_k
                return den + jnp.sum(ee_k, axis=0, keepdims=True)

            den = lax.fori_loop(0, NCK, p2, jnp.zeros((1, B), jnp.float32))

            deninv = 1.0 / jnp.maximum(den, 1e-30)

            def p3(k, r):
                sl = pl.ds(k * CK, CK)
                alpha_k = e_ref[sl] * deninv
                return r + _dot3(alpha_k, feat_ref[0, sl],
                                 (((0,), (0,)), ((), ())))

            r = lax.fori_loop(0, NCK, p3, jnp.zeros((B, D), jnp.float32))
            q_star = jnp.concatenate([q, r], axis=1)
        g = (_dot3(q_star, wsp_ref[...], (((1,), (0,)), ((), ())))
             + bsp_ref[...])
        a = asp_ref[0, 0]
        out_ref[0] = jnp.where(g >= 0, g, a * g)

    return pl.pallas_call(
        body,
        grid=(2,),
        in_specs=[
            pl.BlockSpec((1, NP, 2 * H), lambda g: (g, 0, 0)),
            pl.BlockSpec((1, NP, B), lambda g: (g, 0, 0)),
            pl.BlockSpec((2 * D, 4 * D), lambda g: (0, 0)),
            pl.BlockSpec((1, 4 * D), lambda g: (0, 0)),
            pl.BlockSpec((D, 4 * D), lambda g: (0, 0)),
            pl.BlockSpec((1, 4 * D), lambda g: (0, 0)),
            pl.BlockSpec((2 * D, H), lambda g: (0, 0)),
            pl.BlockSpec((1, H), lambda g: (0, 0)),
            pl.BlockSpec((1, 1), lambda g: (0, 0)),
        ],
        out_specs=pl.BlockSpec((1, B, H), lambda g: (g, 0, 0)),
        out_shape=jax.ShapeDtypeStruct((2, B, H), jnp.float32),
        scratch_shapes=[pltpu.VMEM((NP, B), jnp.float32)],
    )(feat2, gidb, w_ihT, b_ih, w_hhT, b_hh, wsp, bsp, asp)


def _tc_head(g2, wp1, bp1, ap1, wp2, bp2, ap2, wp3p, bp3p):
    def body(g_ref, w1_ref, b1_ref, a1_ref, w2_ref, b2_ref, a2_ref,
             w3_ref, b3_ref, out_ref):
        cat = jnp.concatenate([g_ref[0], g_ref[1]], axis=1)
        h1 = jnp.dot(cat, w1_ref[...], preferred_element_type=jnp.float32, precision=lax.Precision.HIGHEST) + b1_ref[...]
        h1 = jnp.where(h1 >= 0, h1, a1_ref[0, 0] * h1)
        h2 = jnp.dot(h1, w2_ref[...], preferred_element_type=jnp.float32, precision=lax.Precision.HIGHEST) + b2_ref[...]
        h2 = jnp.where(h2 >= 0, h2, a2_ref[0, 0] * h2)
        out_ref[...] = jnp.dot(h2, w3_ref[...], preferred_element_type=jnp.float32, precision=lax.Precision.HIGHEST) + b3_ref[...]

    return pl.pallas_call(
        body,
        grid=(1,),
        in_specs=[
            pl.BlockSpec((2, B, H), lambda i: (0, 0, 0)),
            pl.BlockSpec((2 * H, H), lambda i: (0, 0)),
            pl.BlockSpec((1, H), lambda i: (0, 0)),
            pl.BlockSpec((1, 1), lambda i: (0, 0)),
            pl.BlockSpec((H, H), lambda i: (0, 0)),
            pl.BlockSpec((1, H), lambda i: (0, 0)),
            pl.BlockSpec((1, 1), lambda i: (0, 0)),
            pl.BlockSpec((H, H), lambda i: (0, 0)),
            pl.BlockSpec((1, H), lambda i: (0, 0)),
        ],
        out_specs=pl.BlockSpec((B, H), lambda i: (0, 0)),
        out_shape=jax.ShapeDtypeStruct((B, H), jnp.float32),
    )(g2, wp1, bp1, ap1, wp2, bp2, ap2, wp3p, bp3p)


def _pad_edges(e):
    return jnp.pad(e, (0, EP - E), constant_values=N).astype(jnp.int32)


def kernel(x_r, edge_index_r, edge_attr_r, graph_ids_r,
           x_p, edge_index_p, edge_attr_p, graph_ids_p,
           Wp, bp, Wtag, btag, Wg1, bg1, Wg2, bg2,
           W_ih, W_hh, b_ih, b_hh, Wsp, bsp, asp,
           Wp1, bp1, ap1, Wp2, bp2, ap2, Wp3, bp3):
    f32 = jnp.float32
    esrc = jnp.stack([_pad_edges(edge_index_r[0]), _pad_edges(edge_index_p[0])])
    edst = jnp.stack([_pad_edges(edge_index_r[1]), _pad_edges(edge_index_p[1])])
    esrc_g = esrc + jnp.array([[0], [NP]], jnp.int32)
    x2 = jnp.stack([
        jnp.pad(x_r, ((0, NP - N), (0, 0))),
        jnp.pad(x_p, ((0, NP - N), (0, 0))),
    ]).astype(f32)
    gidb = jnp.broadcast_to(jnp.stack([
        jnp.pad(graph_ids_r, (0, NP - N), constant_values=B),
        jnp.pad(graph_ids_p, (0, NP - N), constant_values=B),
    ]).astype(jnp.int32).reshape(2, NP, 1), (2, NP, B))
    zeros128 = jnp.zeros((CHUNK, H), f32)
    ones2 = jnp.ones((2 * NP, H), f32)
    bp_ = bp.reshape(1, H).astype(f32)
    btag_ = btag.reshape(1, H).astype(f32)
    bg1_ = bg1.reshape(1, H).astype(f32)
    bg2_ = bg2.reshape(1, H).astype(f32)
    w_ihT = W_ih.T.astype(f32)
    w_hhT = W_hh.T.astype(f32)
    b_ih_ = b_ih.reshape(1, -1).astype(f32)
    b_hh_ = b_hh.reshape(1, -1).astype(f32)
    bsp_ = bsp.reshape(1, H).astype(f32)
    asp_ = jnp.asarray(asp, f32).reshape(1, 1)
    bp1_ = bp1.reshape(1, H).astype(f32)
    ap1_ = jnp.asarray(ap1, f32).reshape(1, 1)
    bp2_ = bp2.reshape(1, H).astype(f32)
    ap2_ = jnp.asarray(ap2, f32).reshape(1, 1)
    wp3p = jnp.pad(Wp3, ((0, 0), (0, H - Wp3.shape[1]))).astype(f32)
    bp3p = jnp.pad(bp3, (0, H - bp3.shape[0])).reshape(1, H).astype(f32)

    yo = _sc_prop(ones2, esrc_g, esrc, zeros128)
    yi = _sc_prop(ones2, esrc_g, edst, zeros128)
    douts = yo.reshape(2, NP, H)[:, :, :16]
    dins = yi.reshape(2, NP, H)[:, :, :16]

    h0, u = _tc_prologue(x2, Wp.astype(f32), bp_, douts)
    nf = h0
    hidden = h0

    for _ in range(3):
        y1 = _sc_prop(u.reshape(2 * NP, H), esrc_g, edst, zeros128)
        f1, u2 = _tc_midhop(y1.reshape(2, NP, H), dins, douts)
        y2 = _sc_prop(u2.reshape(2 * NP, H), esrc_g, edst, zeros128)
        nf, hidden, u = _tc_iter_end(
            y2.reshape(2, NP, H), nf, f1, hidden, dins, douts,
            Wtag.astype(f32), btag_, Wg1.astype(f32), bg1_, Wg2.astype(f32), bg2_)

    feat2 = jnp.concatenate([h0, nf], axis=2)
    g2 = _tc_set2set(feat2, gidb, w_ihT, b_ih_, w_hhT, b_hh_,
                     Wsp.astype(f32), bsp_, asp_)
    out = _tc_head(g2, Wp1.astype(f32), bp1_, ap1_, Wp2.astype(f32), bp2_,
                   ap2_, wp3p, bp3p)
    return (out[:, 0], out[:, 1])


# speedup vs baseline: 4.2197x; 1.3594x over previous
import functools
import math

import jax
import jax.numpy as jnp
from jax import lax
from jax.experimental import pallas as pl
from jax.experimental.pallas import tpu as pltpu
from jax.experimental.pallas import tpu_sc as plsc

N = 10000
NP = 10240
E = 320000
EP = 327680
H = 128
B = 64
NT = 16
NC = 2
CHUNK = 128
EPT = EP // NT
NCHUNK = EPT // CHUNK
RPT = NP // NT
WSIG = 1.0 / (1.0 + math.exp(-1.0))

def _mesh():
    return plsc.VectorSubcoreMesh(
        core_axis_name="c", subcore_axis_name="s", num_cores=NC, num_subcores=NT)


def _sc_prop(u, esrc3, edst3, zeros128):
    @functools.partial(
        pl.kernel,
        out_type=jax.ShapeDtypeStruct((2 * NP, H), jnp.float32),
        mesh=_mesh(),
        scratch_types=[
            pltpu.MemorySpace.VMEM_SHARED((NP, H), jnp.float32),
            pltpu.MemorySpace.VMEM((CHUNK, H), jnp.float32),
            pltpu.MemorySpace.VMEM((CHUNK, H), jnp.float32),
            pltpu.MemorySpace.VMEM((CHUNK,), jnp.int32),
            pltpu.MemorySpace.VMEM((CHUNK,), jnp.int32),
            pltpu.MemorySpace.VMEM((CHUNK,), jnp.int32),
            pltpu.MemorySpace.VMEM((CHUNK,), jnp.int32),
            pltpu.SemaphoreType.DMA,
            pltpu.SemaphoreType.DMA,
            pltpu.SemaphoreType.DMA,
            pltpu.SemaphoreType.DMA,
            pltpu.SemaphoreType.DMA,
            pltpu.SemaphoreType.DMA,
        ],
    )
    def prop_kernel(u_hbm, esrc_hbm, edst_hbm, zeros_hbm, y_hbm,
                    acc, buf0, buf1, sidx0, sidx1, didx0, didx1,
                    semg0, semg1, sems0, sems1, semd0, semd1):
        c = lax.axis_index("c")
        s = lax.axis_index("s")
        bufs = (buf0, buf1)
        sidxs = (sidx0, sidx1)
        didxs = (didx0, didx1)
        semg = (semg0, semg1)
        sems = (sems0, sems1)
        semd = (semd0, semd1)
        for b in range(RPT // CHUNK):
            r0 = s * RPT + b * CHUNK
            pltpu.sync_copy(zeros_hbm, acc.at[pl.ds(r0, CHUNK)])
        plsc.subcore_barrier()

        def wait_buf(p):
            pltpu.make_async_copy(zeros_hbm, bufs[p], semg[p]).wait()

        def wait_idx(ibuf, sem):
            pltpu.make_async_copy(edst_hbm.at[0, 0, 0], ibuf, sem).wait()

        pltpu.sync_copy(esrc_hbm.at[c, s, 0], sidx0)
        pltpu.async_copy(u_hbm.at[sidx0], buf0, semg0)
        pltpu.async_copy(edst_hbm.at[c, s, 0], didx0, semd0)
        pltpu.async_copy(esrc_hbm.at[c, s, 1], sidx1, sems1)

        def stage(k, p, fetch_sidx2, fetch_next):
            q = 1 - p
            wait_buf(p)
            wait_idx(didxs[p], semd[p])
            if fetch_sidx2:
                pltpu.async_copy(esrc_hbm.at[c, s, k + 2], sidxs[p], sems[p])
            if fetch_next:
                wait_idx(sidxs[q], sems[q])
                pltpu.async_copy(u_hbm.at[sidxs[q]], bufs[q], semg[q])
                pltpu.async_copy(edst_hbm.at[c, s, k + 1], didxs[q], semd[q])
            pltpu.sync_copy(bufs[p], acc.at[didxs[p]], add=True)

        def pair(j, carry):
            stage(2 * j, 0, True, True)
            stage(2 * j + 1, 1, True, True)
            return carry

        lax.fori_loop(0, NCHUNK // 2 - 1, pair, 0)
        stage(NCHUNK - 2, 0, False, True)
        stage(NCHUNK - 1, 1, False, False)

        plsc.subcore_barrier()
        for b in range(RPT // CHUNK):
            r0 = s * RPT + b * CHUNK
            pltpu.sync_copy(acc.at[pl.ds(r0, CHUNK)],
                            y_hbm.at[pl.ds(c * NP + r0, CHUNK)])

    return prop_kernel(u, esrc3, edst3, zeros128)


_RB = 1280
_NB = NP // _RB


def _scales(cnt):
    return jnp.where(cnt > 0, lax.rsqrt(jnp.maximum(cnt, 1e-30) * WSIG), 0.0)


def _tc_prologue(x2, wp, bp, douts):
    def body(x_ref, wp_ref, bp_ref, do_ref, h0_ref, u_ref):
        h0 = jax.nn.relu(
            jnp.dot(x_ref[0], wp_ref[...], preferred_element_type=jnp.float32, precision=lax.Precision.HIGHEST)
            + bp_ref[...])
        t = WSIG * _scales(do_ref[0][:, 0:1])
        h0_ref[0] = h0
        u_ref[0] = h0 * t

    return pl.pallas_call(
        body,
        grid=(2, _NB),
        in_specs=[
            pl.BlockSpec((1, _RB, H), lambda g, i: (g, i, 0)),
            pl.BlockSpec((H, H), lambda g, i: (0, 0)),
            pl.BlockSpec((1, H), lambda g, i: (0, 0)),
            pl.BlockSpec((1, _RB, 16), lambda g, i: (g, i, 0)),
        ],
        out_specs=[
            pl.BlockSpec((1, _RB, H), lambda g, i: (g, i, 0)),
            pl.BlockSpec((1, _RB, H), lambda g, i: (g, i, 0)),
        ],
        out_shape=[
            jax.ShapeDtypeStruct((2, NP, H), jnp.float32),
            jax.ShapeDtypeStruct((2, NP, H), jnp.float32),
        ],
    )(x2, wp, bp, douts)


def _tc_midhop(y1, dins, douts):
    def body(y_ref, di_ref, do_ref, f1_ref, u2_ref):
        s = _scales(di_ref[0][:, 0:1])
        t = WSIG * _scales(do_ref[0][:, 0:1])
        f1 = y_ref[0] * s
        f1_ref[0] = f1
        u2_ref[0] = f1 * t

    return pl.pallas_call(
        body,
        grid=(2, _NB),
        in_specs=[
            pl.BlockSpec((1, _RB, H), lambda g, i: (g, i, 0)),
            pl.BlockSpec((1, _RB, 16), lambda g, i: (g, i, 0)),
            pl.BlockSpec((1, _RB, 16), lambda g, i: (g, i, 0)),
        ],
        out_specs=[
            pl.BlockSpec((1, _RB, H), lambda g, i: (g, i, 0)),
            pl.BlockSpec((1, _RB, H), lambda g, i: (g, i, 0)),
        ],
        out_shape=[
            jax.ShapeDtypeStruct((2, NP, H), jnp.float32),
            jax.ShapeDtypeStruct((2, NP, H), jnp.float32),
        ],
    )(y1, dins, douts)


def _tc_iter_end(y2, nf, f1, hidden, dins, douts, wtag, btag, wg1, bg1, wg2, bg2):
    def body(y_ref, nf_ref, f1_ref, hid_ref, di_ref, do_ref,
             wtag_ref, btag_ref, wg1_ref, bg1_ref, wg2_ref, bg2_ref,
             nf2_ref, hid2_ref, u_ref):
        s = _scales(di_ref[0][:, 0:1])
        t = WSIG * _scales(do_ref[0][:, 0:1])
        f2 = y_ref[0] * s
        fst = jnp.concatenate([nf_ref[0], f1_ref[0], f2], axis=1)
        tt = jax.nn.relu(
            jnp.dot(fst, wtag_ref[...], preferred_element_type=jnp.float32, precision=lax.Precision.HIGHEST)
            + btag_ref[...])
        cat = jnp.concatenate([tt, hid_ref[0]], axis=1)
        nf2 = jnp.dot(cat, wg1_ref[...], preferred_element_type=jnp.float32, precision=lax.Precision.HIGHEST) + bg1_ref[...]
        hid2 = jnp.dot(cat, wg2_ref[...], preferred_element_type=jnp.float32, precision=lax.Precision.HIGHEST) + bg2_ref[...]
        nf2_ref[0] = nf2
        hid2_ref[0] = hid2
        u_ref[0] = nf2 * t

    return pl.pallas_call(
        body,
        grid=(2, _NB),
        in_specs=[
            pl.BlockSpec((1, _RB, H), lambda g, i: (g, i, 0)),
            pl.BlockSpec((1, _RB, H), lambda g, i: (g, i, 0)),
            pl.BlockSpec((1, _RB, H), lambda g, i: (g, i, 0)),
            pl.BlockSpec((1, _RB, H), lambda g, i: (g, i, 0)),
            pl.BlockSpec((1, _RB, 16), lambda g, i: (g, i, 0)),
            pl.BlockSpec((1, _RB, 16), lambda g, i: (g, i, 0)),
            pl.BlockSpec((3 * H, H), lambda g, i: (0, 0)),
            pl.BlockSpec((1, H), lambda g, i: (0, 0)),
            pl.BlockSpec((2 * H, H), lambda g, i: (0, 0)),
            pl.BlockSpec((1, H), lambda g, i: (0, 0)),
            pl.BlockSpec((2 * H, H), lambda g, i: (0, 0)),
            pl.BlockSpec((1, H), lambda g, i: (0, 0)),
        ],
        out_specs=[
            pl.BlockSpec((1, _RB, H), lambda g, i: (g, i, 0)),
            pl.BlockSpec((1, _RB, H), lambda g, i: (g, i, 0)),
            pl.BlockSpec((1, _RB, H), lambda g, i: (g, i, 0)),
        ],
        out_shape=[
            jax.ShapeDtypeStruct((2, NP, H), jnp.float32),
            jax.ShapeDtypeStruct((2, NP, H), jnp.float32),
            jax.ShapeDtypeStruct((2, NP, H), jnp.float32),
        ],
    )(y2, nf, f1, hidden, dins, douts, wtag, btag, wg1, bg1, wg2, bg2)


def _dot3(a, b, dims):
    f32 = jnp.float32
    a_hi = a.astype(jnp.bfloat16).astype(f32)
    a_lo = a - a_hi
    b_hi = b.astype(jnp.bfloat16).astype(f32)
    b_lo = b - b_hi
    d = lambda x, y: lax.dot_general(x, y, dims, preferred_element_type=f32,
                                     precision=lax.Precision.DEFAULT)
    return d(a_hi, b_hi) + d(a_lo, b_hi) + d(a_hi, b_lo)


def _sigm(x):
    return jax.nn.sigmoid(x)


def _tc_set2set(feat2, gidb, w_ihT, b_ih, w_hhT, b_hh, wsp, bsp, asp):
    D = 2 * H
    NCK = 8
    CK = NP // NCK

    def body(feat_ref, gidb_ref, wih_ref, bih_ref, whh_ref, bhh_ref,
             wsp_ref, bsp_ref, asp_ref, out_ref, e_ref):
        hC = jnp.zeros((B, D), jnp.float32)
        cC = jnp.zeros((B, D), jnp.float32)
        q_star = jnp.zeros((B, 2 * D), jnp.float32)
        for _ in range(3):
            gates = (_dot3(q_star, wih_ref[...], (((1,), (0,)), ((), ())))
                     + bih_ref[...]
                     + _dot3(hC, whh_ref[...], (((1,), (0,)), ((), ())))
                     + bhh_ref[...])
            ig = gates[:, 0:D]
            fg = gates[:, D:2 * D]
            gg = gates[:, 2 * D:3 * D]
            og = gates[:, 3 * D:4 * D]
            cC = _sigm(fg) * cC + _sigm(ig) * jnp.tanh(gg)
            hC = _sigm(og) * jnp.tanh(cC)
            q = hC
            cols = lax.broadcasted_iota(jnp.int32, (CK, B), 1)

            def p1(k, emax):
                sl = pl.ds(k * CK, CK)
                oh = gidb_ref[0, sl] == cols
                e_k = _dot3(feat_ref[0, sl], q, (((1,), (1,)), ((), ())))
                e_ref[sl] = e_k
                return jnp.maximum(
                    emax, jnp.max(jnp.where(oh, e_k, -3e38), axis=0,
                                  keepdims=True))

            emax = lax.fori_loop(0, NCK, p1, jnp.full((1, B), -3e38, jnp.float32))

            def p2(k, den):
                sl = pl.ds(k * CK, CK)
                oh = gidb_ref[0, sl] == cols
                ee_k = jnp.where(oh, jnp.exp(e_ref[sl] - emax), 0.0)
                e_ref[sl] = ee_k
                return den + jnp.sum(ee_k, axis=0, keepdims=True)

            den = lax.fori_loop(0, NCK, p2, jnp.zeros((1, B), jnp.float32))

            deninv = 1.0 / jnp.maximum(den, 1e-30)

            def p3(k, r):
                sl = pl.ds(k * CK, CK)
                alpha_k = e_ref[sl] * deninv
                return r + _dot3(alpha_k, feat_ref[0, sl],
                                 (((0,), (0,)), ((), ())))

            r = lax.fori_loop(0, NCK, p3, jnp.zeros((B, D), jnp.float32))
            q_star = jnp.concatenate([q, r], axis=1)
        g = (_dot3(q_star, wsp_ref[...], (((1,), (0,)), ((), ())))
             + bsp_ref[...])
        a = asp_ref[0, 0]
        out_ref[0] = jnp.where(g >= 0, g, a * g)

    return pl.pallas_call(
        body,
        grid=(2,),
        in_specs=[
            pl.BlockSpec((1, NP, 2 * H), lambda g: (g, 0, 0)),
            pl.BlockSpec((1, NP, B), lambda g: (g, 0, 0)),
            pl.BlockSpec((2 * D, 4 * D), lambda g: (0, 0)),
            pl.BlockSpec((1, 4 * D), lambda g: (0, 0)),
            pl.BlockSpec((D, 4 * D), lambda g: (0, 0)),
            pl.BlockSpec((1, 4 * D), lambda g: (0, 0)),
            pl.BlockSpec((2 * D, H), lambda g: (0, 0)),
            pl.BlockSpec((1, H), lambda g: (0, 0)),
            pl.BlockSpec((1, 1), lambda g: (0, 0)),
        ],
        out_specs=pl.BlockSpec((1, B, H), lambda g: (g, 0, 0)),
        out_shape=jax.ShapeDtypeStruct((2, B, H), jnp.float32),
        scratch_shapes=[pltpu.VMEM((NP, B), jnp.float32)],
    )(feat2, gidb, w_ihT, b_ih, w_hhT, b_hh, wsp, bsp, asp)


def _tc_head(g2, wp1, bp1, ap1, wp2, bp2, ap2, wp3p, bp3p):
    def body(g_ref, w1_ref, b1_ref, a1_ref, w2_ref, b2_ref, a2_ref,
             w3_ref, b3_ref, out_ref):
        cat = jnp.concatenate([g_ref[0], g_ref[1]], axis=1)
        h1 = jnp.dot(cat, w1_ref[...], preferred_element_type=jnp.float32, precision=lax.Precision.HIGHEST) + b1_ref[...]
        h1 = jnp.where(h1 >= 0, h1, a1_ref[0, 0] * h1)
        h2 = jnp.dot(h1, w2_ref[...], preferred_element_type=jnp.float32, precision=lax.Precision.HIGHEST) + b2_ref[...]
        h2 = jnp.where(h2 >= 0, h2, a2_ref[0, 0] * h2)
        out_ref[...] = jnp.dot(h2, w3_ref[...], preferred_element_type=jnp.float32, precision=lax.Precision.HIGHEST) + b3_ref[...]

    return pl.pallas_call(
        body,
        grid=(1,),
        in_specs=[
            pl.BlockSpec((2, B, H), lambda i: (0, 0, 0)),
            pl.BlockSpec((2 * H, H), lambda i: (0, 0)),
            pl.BlockSpec((1, H), lambda i: (0, 0)),
            pl.BlockSpec((1, 1), lambda i: (0, 0)),
            pl.BlockSpec((H, H), lambda i: (0, 0)),
            pl.BlockSpec((1, H), lambda i: (0, 0)),
            pl.BlockSpec((1, 1), lambda i: (0, 0)),
            pl.BlockSpec((H, H), lambda i: (0, 0)),
            pl.BlockSpec((1, H), lambda i: (0, 0)),
        ],
        out_specs=pl.BlockSpec((B, H), lambda i: (0, 0)),
        out_shape=jax.ShapeDtypeStruct((B, H), jnp.float32),
    )(g2, wp1, bp1, ap1, wp2, bp2, ap2, wp3p, bp3p)


def _pad_edges(e):
    return jnp.pad(e, (0, EP - E), constant_values=N).astype(jnp.int32)


def kernel(x_r, edge_index_r, edge_attr_r, graph_ids_r,
           x_p, edge_index_p, edge_attr_p, graph_ids_p,
           Wp, bp, Wtag, btag, Wg1, bg1, Wg2, bg2,
           W_ih, W_hh, b_ih, b_hh, Wsp, bsp, asp,
           Wp1, bp1, ap1, Wp2, bp2, ap2, Wp3, bp3):
    f32 = jnp.float32
    esrc = jnp.stack([_pad_edges(edge_index_r[0]), _pad_edges(edge_index_p[0])])
    edst = jnp.stack([_pad_edges(edge_index_r[1]), _pad_edges(edge_index_p[1])])
    esrc_g = ((esrc + jnp.array([[0], [NP]], jnp.int32))
              .reshape(2, NT, NCHUNK, CHUNK))
    esrc4 = esrc.reshape(2, NT, NCHUNK, CHUNK)
    edst4 = edst.reshape(2, NT, NCHUNK, CHUNK)
    x2 = jnp.stack([
        jnp.pad(x_r, ((0, NP - N), (0, 0))),
        jnp.pad(x_p, ((0, NP - N), (0, 0))),
    ]).astype(f32)
    gidb = jnp.broadcast_to(jnp.stack([
        jnp.pad(graph_ids_r, (0, NP - N), constant_values=B),
        jnp.pad(graph_ids_p, (0, NP - N), constant_values=B),
    ]).astype(jnp.int32).reshape(2, NP, 1), (2, NP, B))
    zeros128 = jnp.zeros((CHUNK, H), f32)
    ones2 = jnp.ones((2 * NP, H), f32)
    bp_ = bp.reshape(1, H).astype(f32)
    btag_ = btag.reshape(1, H).astype(f32)
    bg1_ = bg1.reshape(1, H).astype(f32)
    bg2_ = bg2.reshape(1, H).astype(f32)
    w_ihT = W_ih.T.astype(f32)
    w_hhT = W_hh.T.astype(f32)
    b_ih_ = b_ih.reshape(1, -1).astype(f32)
    b_hh_ = b_hh.reshape(1, -1).astype(f32)
    bsp_ = bsp.reshape(1, H).astype(f32)
    asp_ = jnp.asarray(asp, f32).reshape(1, 1)
    bp1_ = bp1.reshape(1, H).astype(f32)
    ap1_ = jnp.asarray(ap1, f32).reshape(1, 1)
    bp2_ = bp2.reshape(1, H).astype(f32)
    ap2_ = jnp.asarray(ap2, f32).reshape(1, 1)
    wp3p = jnp.pad(Wp3, ((0, 0), (0, H - Wp3.shape[1]))).astype(f32)
    bp3p = jnp.pad(bp3, (0, H - bp3.shape[0])).reshape(1, H).astype(f32)

    yo = _sc_prop(ones2, esrc_g, esrc4, zeros128)
    yi = _sc_prop(ones2, esrc_g, edst4, zeros128)
    douts = yo.reshape(2, NP, H)[:, :, :16]
    dins = yi.reshape(2, NP, H)[:, :, :16]

    h0, u = _tc_prologue(x2, Wp.astype(f32), bp_, douts)
    nf = h0
    hidden = h0

    for _ in range(3):
        y1 = _sc_prop(u.reshape(2 * NP, H), esrc_g, edst4, zeros128)
        f1, u2 = _tc_midhop(y1.reshape(2, NP, H), dins, douts)
        y2 = _sc_prop(u2.reshape(2 * NP, H), esrc_g, edst4, zeros128)
        nf, hidden, u = _tc_iter_end(
            y2.reshape(2, NP, H), nf, f1, hidden, dins, douts,
            Wtag.astype(f32), btag_, Wg1.astype(f32), bg1_, Wg2.astype(f32), bg2_)

    feat2 = jnp.concatenate([h0, nf], axis=2)
    g2 = _tc_set2set(feat2, gidb, w_ihT, b_ih_, w_hhT, b_hh_,
                     Wsp.astype(f32), bsp_, asp_)
    out = _tc_head(g2, Wp1.astype(f32), bp1_, ap1_, Wp2.astype(f32), bp2_,
                   ap2_, wp3p, bp3p)
    return (out[:, 0], out[:, 1])
